# baseline (speedup 1.0000x reference)
import functools

import jax
import jax.numpy as jnp
import numpy as np
from jax.experimental import pallas as pl
from jax.experimental.pallas import tpu as pltpu

EPS = 1e-5


def _knn_kernel(k, cent_ref, pts_ref, idx_ref, d_ref):
    M, N = d_ref.shape
    c = cent_ref[...]
    cx, cy, cz = c[:, 0:1], c[:, 1:2], c[:, 2:3]
    px, py, pz = pts_ref[0:1, :], pts_ref[1:2, :], pts_ref[2:3, :]
    cn2 = cx * cx + cy * cy + cz * cz
    pn2 = px * px + py * py + pz * pz
    def r(v):
        return v.astype(jnp.bfloat16).astype(jnp.float32)
    cross = r(cx) * r(px) + r(cy) * r(py) + r(cz) * r(pz)
    d_ref[...] = (cn2 + pn2) - 2.0 * cross

    lane = jax.lax.broadcasted_iota(jnp.int32, (M, N), 1)
    olane = jax.lax.broadcasted_iota(jnp.int32, (M, k), 1)
    big_i = jnp.int32(2 ** 30)
    inf = jnp.float32(np.inf)

    def body(it, _):
        d = d_ref[...]
        rowmin = jnp.min(d, axis=1, keepdims=True)
        cand = jnp.where(d == rowmin, lane, big_i)
        amin = jnp.min(cand, axis=1, keepdims=True)
        idx_ref[...] = jnp.where(olane == it, amin, idx_ref[...])
        d_ref[...] = jnp.where(cand == amin, inf, d)
        return 0

    jax.lax.fori_loop(0, k, body, 0)


def _knn_idx(cent, pts, k):
    B, M, _ = cent.shape
    N = pts.shape[1]
    return pl.pallas_call(
        functools.partial(_knn_kernel, k),
        grid=(B,),
        in_specs=[
            pl.BlockSpec((None, M, 3), lambda b: (b, 0, 0)),
            pl.BlockSpec((None, 3, N), lambda b: (b, 0, 0)),
        ],
        out_specs=pl.BlockSpec((None, M, k), lambda b: (b, 0, 0)),
        out_shape=jax.ShapeDtypeStruct((B, M, k), jnp.int32),
        scratch_shapes=[pltpu.VMEM((M, N), jnp.float32)],
    )(cent, pts.transpose(0, 2, 1))


def _bn_scale(pp, i):
    s = pp['g%d' % i] / jnp.sqrt(1.0 + EPS)
    return s, pp['be%d' % i]


def _sa_stage(xyz, feats, sel, nsample, pp):
    if feats is not None:
        pf = jnp.concatenate([xyz, feats], axis=-1)
    else:
        pf = xyz
    cent = jnp.take(xyz, sel, axis=1)
    idx = _knn_idx(cent, xyz, nsample)
    g = jnp.take_along_axis(pf[:, None, :, :], idx[:, :, :, None], axis=2)
    g = g.at[..., :3].add(-cent[:, :, None, :])

    s1, be1 = _bn_scale(pp, 1)
    s2, be2 = _bn_scale(pp, 2)
    s3, be3 = _bn_scale(pp, 3)
    h = jnp.dot(g.reshape(-1, g.shape[-1]), pp['W1'].T) + pp['b1']
    h = jnp.maximum(h * s1 + be1, 0.0)
    h = jnp.dot(h, pp['W2'].T) + pp['b2']
    h = jnp.maximum(h * s2 + be2, 0.0)
    h = jnp.dot(h, pp['W3'].T) + pp['b3']
    h = h * s3 + be3
    h = h.reshape(g.shape[0], g.shape[1], g.shape[2], -1)
    return cent, jnp.max(h, axis=2)


def _head_kernel(f_ref, w1_ref, a1_ref, w2_ref, a2_ref, w3_ref, b3_ref, o_ref):
    bf = jnp.bfloat16
    f = f_ref[...]
    h = jnp.dot(f.astype(bf), w1_ref[...].astype(bf), preferred_element_type=jnp.float32)
    h = jnp.maximum(h * a1_ref[0:1, :] + a1_ref[1:2, :], 0.0)
    h = jnp.dot(h.astype(bf), w2_ref[...].astype(bf), preferred_element_type=jnp.float32)
    h = jnp.maximum(h * a2_ref[0:1, :] + a2_ref[1:2, :], 0.0)
    o_ref[...] = jnp.dot(h.astype(bf), w3_ref[...].astype(bf), preferred_element_type=jnp.float32) + b3_ref[...]


def _head(f, params):
    s1 = params['bn1g'] / jnp.sqrt(1.0 + EPS)
    a1 = jnp.stack([s1, params['fc1b'] * s1 + params['bn1b']])
    s2 = params['bn2g'] / jnp.sqrt(1.0 + EPS)
    a2 = jnp.stack([s2, params['fc2b'] * s2 + params['bn2b']])
    return pl.pallas_call(
        _head_kernel,
        out_shape=jax.ShapeDtypeStruct((f.shape[0], 12), jnp.float32),
    )(f, params['fc1W'].T, a1, params['fc2W'].T, a2, params['fc3W'].T,
      params['fc3b'][None, :])


def kernel(x, params):
    xyz = x[:, :, :3]
    feats = x[:, :, 3:]
    k1, k2 = jax.random.split(jax.random.key(42), 2)
    sel1 = jax.random.permutation(k1, 4096)[:512]
    sel2 = jax.random.permutation(k2, 512)[:128]

    xyz1, p1 = _sa_stage(xyz, feats, sel1, 32, params['sa1'])
    xyz2, p2 = _sa_stage(xyz1, p1, sel2, 64, params['sa2'])

    pp = params['sa3']
    B, M, C = p2.shape
    t = jnp.concatenate([jnp.zeros((B, M, 3), p2.dtype), p2], axis=-1)
    s1, be1 = _bn_scale(pp, 1)
    s2, be2 = _bn_scale(pp, 2)
    s3, be3 = _bn_scale(pp, 3)
    h = jnp.dot(t.reshape(-1, C + 3), pp['W1'].T) + pp['b1']
    h = jnp.maximum(h * s1 + be1, 0.0)
    h = jnp.dot(h, pp['W2'].T) + pp['b2']
    h = jnp.maximum(h * s2 + be2, 0.0)
    h = jnp.dot(h, pp['W3'].T) + pp['b3']
    h = h * s3 + be3
    f = jnp.max(h.reshape(B, M, -1), axis=1)
    return _head(f, params)


# speedup vs baseline: 4.7166x; 4.7166x over previous
import functools

import jax
import jax.numpy as jnp
import numpy as np
from jax.experimental import pallas as pl
from jax.experimental.pallas import tpu as pltpu
from jax.experimental.pallas import tpu_sc as plsc

EPS = 1e-5
BF = jnp.bfloat16


def _knn_kernel(k, nbase, cent_ref, pts_ref, idx_ref, d_ref):
    M, N = d_ref.shape
    c = cent_ref[...]
    cx, cy, cz = c[:, 0:1], c[:, 1:2], c[:, 2:3]
    px, py, pz = pts_ref[0:1, :], pts_ref[1:2, :], pts_ref[2:3, :]
    cn2 = cx * cx + cy * cy + cz * cz
    pn2 = px * px + py * py + pz * pz

    def r(v):
        return v.astype(BF).astype(jnp.float32)
    cross = r(cx) * r(px) + r(cy) * r(py) + r(cz) * r(pz)
    d_ref[...] = (cn2 + pn2) - 2.0 * cross

    lane = jax.lax.broadcasted_iota(jnp.int32, (M, N), 1)
    olane = jax.lax.broadcasted_iota(jnp.int32, (M, k), 1)
    big_i = jnp.int32(2 ** 30)
    inf = jnp.float32(np.inf)
    base = pl.program_id(0) * nbase

    def body(it, _):
        d = d_ref[...]
        rowmin = jnp.min(d, axis=1, keepdims=True)
        cand = jnp.where(d == rowmin, lane, big_i)
        amin = jnp.min(cand, axis=1, keepdims=True)
        idx_ref[...] = jnp.where(olane == it, amin + base, idx_ref[...])
        d_ref[...] = jnp.where(cand == amin, inf, d)
        return 0

    jax.lax.fori_loop(0, k, body, 0)


def _knn_idx(cent, pts, k):
    B, M, _ = cent.shape
    N = pts.shape[1]
    return pl.pallas_call(
        functools.partial(_knn_kernel, k, N),
        grid=(B,),
        in_specs=[
            pl.BlockSpec((None, M, 3), lambda b: (b, 0, 0)),
            pl.BlockSpec((None, 3, N), lambda b: (b, 0, 0)),
        ],
        out_specs=pl.BlockSpec((None, M, k), lambda b: (b, 0, 0)),
        out_shape=jax.ShapeDtypeStruct((B, M, k), jnp.int32),
        scratch_shapes=[pltpu.VMEM((M, N), jnp.float32)],
    )(cent, pts.transpose(0, 2, 1))


def _sc_gather(table, idx, group):
    V, D = table.shape
    R = idx.shape[0]
    NW = 32
    rpw = R // NW
    nch = rpw // 128
    ngrp = nch // group
    rows_g = group * 128
    idx2 = idx.reshape(NW, nch, 128)
    mesh = plsc.VectorSubcoreMesh(core_axis_name="c", subcore_axis_name="s")

    @functools.partial(
        pl.kernel,
        out_type=jax.ShapeDtypeStruct((R, D), jnp.float32),
        mesh=mesh,
        compiler_params=pltpu.CompilerParams(use_tc_tiling_on_sc=False),
        scratch_types=[
            pltpu.VMEM((nch, 128), jnp.int32),
            pltpu.VMEM((rows_g, D), jnp.float32),
            pltpu.VMEM((rows_g, D), jnp.float32),
            pltpu.SemaphoreType.DMA,
            pltpu.SemaphoreType.DMA,
        ],
    )
    def k(table_hbm, idx_hbm, out_hbm, idx_v, b0, b1, s0, s1):
        wid = jax.lax.axis_index("s") * 2 + jax.lax.axis_index("c")
        base = wid * rpw
        pltpu.sync_copy(idx_hbm.at[wid], idx_v)
        bufs = (b0, b1)
        sems = (s0, s1)
        pend = [None, None]

        def fire(gi, slot):
            cps = []
            for j in range(group):
                cps.append(pltpu.async_copy(
                    table_hbm.at[idx_v.at[gi * group + j]],
                    bufs[slot].at[pl.ds(j * 128, 128)],
                    sems[slot]))
            pend[slot] = cps

        def drain(gi, slot):
            for cp in pend[slot]:
                cp.wait()
            pltpu.sync_copy(bufs[slot],
                            out_hbm.at[pl.ds(base + gi * rows_g, rows_g)])

        fire(0, 0)
        for gi in range(1, ngrp):
            fire(gi, gi % 2)
            drain(gi - 1, (gi - 1) % 2)
        drain(ngrp - 1, (ngrp - 1) % 2)

    return k(table, idx2)


def _mlp_kernel(kn, g_ref, cent_ref, w1_ref, a1_ref, w2_ref, a2_ref,
                w3_ref, a3_ref, o_ref):
    MB = cent_ref.shape[0]
    D = g_ref.shape[1]
    t = g_ref[...] - jnp.broadcast_to(
        cent_ref[...][:, None, :], (MB, kn, D)).reshape(MB * kn, D)

    def layer(h, w_ref, a_ref, relu):
        h = jnp.dot(h.astype(BF), w_ref[...].astype(BF),
                    preferred_element_type=jnp.float32)
        h = h + a_ref[0:1, :]
        h = h * a_ref[1:2, :] + a_ref[2:3, :]
        return jnp.maximum(h, 0.0) if relu else h

    h = layer(t, w1_ref, a1_ref, True)
    h = layer(h, w2_ref, a2_ref, True)
    h = layer(h, w3_ref, a3_ref, False)
    C = h.shape[1]
    o_ref[...] = jnp.max(h.reshape(MB, kn, C), axis=1)


def _mlp_max(g, cent, kn, mb, wa):
    B, M, D = cent.shape
    C3 = wa[4].shape[1]
    nmb = M // mb
    specs = [
        pl.BlockSpec((None, mb * kn, D), lambda b, m: (b, m, 0)),
        pl.BlockSpec((None, mb, D), lambda b, m: (b, m, 0)),
    ]
    for w in wa:
        specs.append(pl.BlockSpec(w.shape, lambda b, m: (0,) * w.ndim))
    return pl.pallas_call(
        functools.partial(_mlp_kernel, kn),
        grid=(B, nmb),
        in_specs=specs,
        out_specs=pl.BlockSpec((None, mb, C3), lambda b, m: (b, m, 0)),
        out_shape=jax.ShapeDtypeStruct((B, M, C3), jnp.float32),
    )(g, cent, *wa)


def _tail_kernel(B, M, t_ref, w1_ref, a1_ref, w2_ref, a2_ref, w3_ref, a3_ref,
                 f1_ref, g1_ref, f2_ref, g2_ref, f3_ref, b3_ref, o_ref):
    def layer(h, w_ref, a_ref, relu):
        h = jnp.dot(h.astype(BF), w_ref[...].astype(BF),
                    preferred_element_type=jnp.float32)
        h = h + a_ref[0:1, :]
        h = h * a_ref[1:2, :] + a_ref[2:3, :]
        return jnp.maximum(h, 0.0) if relu else h

    h = layer(t_ref[...], w1_ref, a1_ref, True)
    h = layer(h, w2_ref, a2_ref, True)
    h = layer(h, w3_ref, a3_ref, False)
    C = h.shape[1]
    f = jnp.max(h.reshape(B, M, C), axis=1)

    def fc(h, w_ref, a_ref):
        h = jnp.dot(h.astype(BF), w_ref[...].astype(BF),
                    preferred_element_type=jnp.float32)
        h = h + a_ref[0:1, :]
        return jnp.maximum(h * a_ref[1:2, :] + a_ref[2:3, :], 0.0)

    f = fc(f, f1_ref, g1_ref)
    f = fc(f, f2_ref, g2_ref)
    o_ref[...] = jnp.dot(f.astype(BF), f3_ref[...].astype(BF),
                         preferred_element_type=jnp.float32) + b3_ref[...]


def _tail(p2, params):
    B, M, C = p2.shape
    t = jnp.concatenate([jnp.zeros((B, M, 3), p2.dtype), p2], axis=-1)
    t = t.reshape(B * M, C + 3)
    pp = params['sa3']
    wa = _conv_args(pp)
    s1 = params['bn1g'] / jnp.sqrt(1.0 + EPS)
    g1 = jnp.stack([params['fc1b'], s1, params['bn1b']])
    s2 = params['bn2g'] / jnp.sqrt(1.0 + EPS)
    g2 = jnp.stack([params['fc2b'], s2, params['bn2b']])
    return pl.pallas_call(
        functools.partial(_tail_kernel, B, M),
        out_shape=jax.ShapeDtypeStruct((B, 12), jnp.float32),
    )(t, wa[0], wa[1], wa[2], wa[3], wa[4], wa[5],
      params['fc1W'].T, g1, params['fc2W'].T, g2, params['fc3W'].T,
      params['fc3b'][None, :])


def _conv_args(pp, pad_in=None):
    out = []
    for i in (1, 2, 3):
        W = pp['W%d' % i].T
        if i == 1 and pad_in is not None and W.shape[0] < pad_in:
            W = jnp.concatenate(
                [W, jnp.zeros((pad_in - W.shape[0], W.shape[1]), W.dtype)], 0)
        s = pp['g%d' % i] / jnp.sqrt(1.0 + EPS)
        out.append(W)
        out.append(jnp.stack([pp['b%d' % i], s, pp['be%d' % i]]))
    return out


def _sa_stage(xyz, feats, sel, nsample, pp, dpad, mb, group):
    B, N, _ = xyz.shape
    cols = [xyz] if feats is None else [xyz, feats]
    cin = sum(c.shape[-1] for c in cols)
    if cin < dpad:
        cols.append(jnp.zeros((B, N, dpad - cin), xyz.dtype))
    pf = jnp.concatenate(cols, axis=-1)
    cent = jnp.take(xyz, sel, axis=1)
    M = cent.shape[1]
    idx = _knn_idx(cent, xyz, nsample)
    g = _sc_gather(pf.reshape(B * N, dpad), idx.reshape(-1), group)
    g = g.reshape(B, M * nsample, dpad)
    centp = jnp.concatenate(
        [cent, jnp.zeros((B, M, dpad - 3), cent.dtype)], axis=-1)
    wa = _conv_args(pp, pad_in=dpad)
    return cent, _mlp_max(g, centp, nsample, mb, wa)


def kernel(x, params):
    xyz = x[:, :, :3]
    feats = x[:, :, 3:]
    k1, k2 = jax.random.split(jax.random.key(42), 2)
    sel1 = jax.random.permutation(k1, 4096)[:512]
    sel2 = jax.random.permutation(k2, 512)[:128]

    xyz1, p1 = _sa_stage(xyz, feats, sel1, 32, params['sa1'],
                         dpad=16, mb=128, group=8)
    xyz2, p2 = _sa_stage(xyz1, p1, sel2, 64, params['sa2'],
                         dpad=144, mb=32, group=2)
    return _tail(p2, params)


# speedup vs baseline: 5.0898x; 1.0791x over previous
import functools

import jax
import jax.numpy as jnp
import numpy as np
from jax.experimental import pallas as pl
from jax.experimental.pallas import tpu as pltpu
from jax.experimental.pallas import tpu_sc as plsc

EPS = 1e-5
BF = jnp.bfloat16


def _knn_kernel(k, nbase, cent_ref, pts_ref, idx_ref, d_ref):
    M, N = d_ref.shape
    c = cent_ref[...]
    cx, cy, cz = c[:, 0:1], c[:, 1:2], c[:, 2:3]
    px, py, pz = pts_ref[0:1, :], pts_ref[1:2, :], pts_ref[2:3, :]
    cn2 = cx * cx + cy * cy + cz * cz
    pn2 = px * px + py * py + pz * pz

    def r(v):
        return v.astype(BF).astype(jnp.float32)
    cross = r(cx) * r(px) + r(cy) * r(py) + r(cz) * r(pz)
    d_ref[...] = (cn2 + pn2) - 2.0 * cross

    lane = jax.lax.broadcasted_iota(jnp.int32, (M, N), 1)
    olane = jax.lax.broadcasted_iota(jnp.int32, (M, k), 1)
    big_i = jnp.int32(2 ** 30)
    inf = jnp.float32(np.inf)
    base = pl.program_id(0) * nbase

    UNROLL = 4
    assert k % UNROLL == 0

    def body(it, _):
        d = d_ref[...]
        out = idx_ref[...]
        for u in range(UNROLL):
            rowmin = jnp.min(d, axis=1, keepdims=True)
            cand = jnp.where(d == rowmin, lane, big_i)
            amin = jnp.min(cand, axis=1, keepdims=True)
            out = jnp.where(olane == it * UNROLL + u, amin + base, out)
            d = jnp.where(cand == amin, inf, d)
        idx_ref[...] = out
        d_ref[...] = d
        return 0

    jax.lax.fori_loop(0, k // UNROLL, body, 0)


def _knn_idx(cent, pts, k):
    B, M, _ = cent.shape
    N = pts.shape[1]
    return pl.pallas_call(
        functools.partial(_knn_kernel, k, N),
        grid=(B,),
        in_specs=[
            pl.BlockSpec((None, M, 3), lambda b: (b, 0, 0)),
            pl.BlockSpec((None, 3, N), lambda b: (b, 0, 0)),
        ],
        out_specs=pl.BlockSpec((None, M, k), lambda b: (b, 0, 0)),
        out_shape=jax.ShapeDtypeStruct((B, M, k), jnp.int32),
        scratch_shapes=[pltpu.VMEM((M, N), jnp.float32)],
    )(cent, pts.transpose(0, 2, 1))


def _sc_gather(table, idx, group):
    V, D = table.shape
    R = idx.shape[0]
    NW = 32
    rpw = R // NW
    nch = rpw // 128
    ngrp = nch // group
    rows_g = group * 128
    idx2 = idx.reshape(NW, nch, 128)
    mesh = plsc.VectorSubcoreMesh(core_axis_name="c", subcore_axis_name="s")

    @functools.partial(
        pl.kernel,
        out_type=jax.ShapeDtypeStruct((R, D), jnp.float32),
        mesh=mesh,
        compiler_params=pltpu.CompilerParams(use_tc_tiling_on_sc=False),
        scratch_types=[
            pltpu.VMEM((nch, 128), jnp.int32),
            pltpu.VMEM((rows_g, D), jnp.float32),
            pltpu.VMEM((rows_g, D), jnp.float32),
            pltpu.SemaphoreType.DMA,
            pltpu.SemaphoreType.DMA,
        ],
    )
    def k(table_hbm, idx_hbm, out_hbm, idx_v, b0, b1, s0, s1):
        wid = jax.lax.axis_index("s") * 2 + jax.lax.axis_index("c")
        base = wid * rpw
        pltpu.sync_copy(idx_hbm.at[wid], idx_v)
        bufs = (b0, b1)
        sems = (s0, s1)
        pend = [None, None]

        def fire(gi, slot):
            cps = []
            for j in range(group):
                cps.append(pltpu.async_copy(
                    table_hbm.at[idx_v.at[gi * group + j]],
                    bufs[slot].at[pl.ds(j * 128, 128)],
                    sems[slot]))
            pend[slot] = cps

        def drain(gi, slot):
            for cp in pend[slot]:
                cp.wait()
            pltpu.sync_copy(bufs[slot],
                            out_hbm.at[pl.ds(base + gi * rows_g, rows_g)])

        fire(0, 0)
        for gi in range(1, ngrp):
            fire(gi, gi % 2)
            drain(gi - 1, (gi - 1) % 2)
        drain(ngrp - 1, (ngrp - 1) % 2)

    return k(table, idx2)


def _mlp_kernel(kn, g_ref, cent_ref, w1_ref, a1_ref, w2_ref, a2_ref,
                w3_ref, a3_ref, o_ref):
    MB = cent_ref.shape[0]
    D = g_ref.shape[1]
    t = g_ref[...] - jnp.broadcast_to(
        cent_ref[...][:, None, :], (MB, kn, D)).reshape(MB * kn, D)

    def layer(h, w_ref, a_ref, relu):
        h = jnp.dot(h.astype(BF), w_ref[...].astype(BF),
                    preferred_element_type=jnp.float32)
        h = h + a_ref[0:1, :]
        h = h * a_ref[1:2, :] + a_ref[2:3, :]
        return jnp.maximum(h, 0.0) if relu else h

    h = layer(t, w1_ref, a1_ref, True)
    h = layer(h, w2_ref, a2_ref, True)
    h = layer(h, w3_ref, a3_ref, False)
    C = h.shape[1]
    o_ref[...] = jnp.max(h.reshape(MB, kn, C), axis=1)


def _mlp_max(g, cent, kn, mb, wa):
    B, M, D = cent.shape
    C3 = wa[4].shape[1]
    nmb = M // mb
    specs = [
        pl.BlockSpec((None, mb * kn, D), lambda b, m: (b, m, 0)),
        pl.BlockSpec((None, mb, D), lambda b, m: (b, m, 0)),
    ]
    for w in wa:
        specs.append(pl.BlockSpec(w.shape, lambda b, m: (0,) * w.ndim))
    return pl.pallas_call(
        functools.partial(_mlp_kernel, kn),
        grid=(B, nmb),
        in_specs=specs,
        out_specs=pl.BlockSpec((None, mb, C3), lambda b, m: (b, m, 0)),
        out_shape=jax.ShapeDtypeStruct((B, M, C3), jnp.float32),
    )(g, cent, *wa)


def _tail_kernel(B, M, t_ref, w1_ref, a1_ref, w2_ref, a2_ref, w3_ref, a3_ref,
                 f1_ref, g1_ref, f2_ref, g2_ref, f3_ref, b3_ref, o_ref):
    def layer(h, w_ref, a_ref, relu):
        h = jnp.dot(h.astype(BF), w_ref[...].astype(BF),
                    preferred_element_type=jnp.float32)
        h = h + a_ref[0:1, :]
        h = h * a_ref[1:2, :] + a_ref[2:3, :]
        return jnp.maximum(h, 0.0) if relu else h

    h = layer(t_ref[...], w1_ref, a1_ref, True)
    h = layer(h, w2_ref, a2_ref, True)
    h = layer(h, w3_ref, a3_ref, False)
    C = h.shape[1]
    f = jnp.max(h.reshape(B, M, C), axis=1)

    def fc(h, w_ref, a_ref):
        h = jnp.dot(h.astype(BF), w_ref[...].astype(BF),
                    preferred_element_type=jnp.float32)
        h = h + a_ref[0:1, :]
        return jnp.maximum(h * a_ref[1:2, :] + a_ref[2:3, :], 0.0)

    f = fc(f, f1_ref, g1_ref)
    f = fc(f, f2_ref, g2_ref)
    o_ref[...] = jnp.dot(f.astype(BF), f3_ref[...].astype(BF),
                         preferred_element_type=jnp.float32) + b3_ref[...]


def _tail(p2, params):
    B, M, C = p2.shape
    t = jnp.concatenate([jnp.zeros((B, M, 3), p2.dtype), p2], axis=-1)
    t = t.reshape(B * M, C + 3)
    pp = params['sa3']
    wa = _conv_args(pp)
    s1 = params['bn1g'] / jnp.sqrt(1.0 + EPS)
    g1 = jnp.stack([params['fc1b'], s1, params['bn1b']])
    s2 = params['bn2g'] / jnp.sqrt(1.0 + EPS)
    g2 = jnp.stack([params['fc2b'], s2, params['bn2b']])
    return pl.pallas_call(
        functools.partial(_tail_kernel, B, M),
        out_shape=jax.ShapeDtypeStruct((B, 12), jnp.float32),
    )(t, wa[0], wa[1], wa[2], wa[3], wa[4], wa[5],
      params['fc1W'].T, g1, params['fc2W'].T, g2, params['fc3W'].T,
      params['fc3b'][None, :])


def _conv_args(pp, pad_in=None):
    out = []
    for i in (1, 2, 3):
        W = pp['W%d' % i].T
        if i == 1 and pad_in is not None and W.shape[0] < pad_in:
            W = jnp.concatenate(
                [W, jnp.zeros((pad_in - W.shape[0], W.shape[1]), W.dtype)], 0)
        s = pp['g%d' % i] / jnp.sqrt(1.0 + EPS)
        out.append(W)
        out.append(jnp.stack([pp['b%d' % i], s, pp['be%d' % i]]))
    return out


def _sa_stage(xyz, feats, sel, nsample, pp, dpad, mb, group):
    B, N, _ = xyz.shape
    cols = [xyz] if feats is None else [xyz, feats]
    cin = sum(c.shape[-1] for c in cols)
    if cin < dpad:
        cols.append(jnp.zeros((B, N, dpad - cin), xyz.dtype))
    pf = jnp.concatenate(cols, axis=-1)
    cent = jnp.take(xyz, sel, axis=1)
    M = cent.shape[1]
    idx = _knn_idx(cent, xyz, nsample)
    g = _sc_gather(pf.reshape(B * N, dpad), idx.reshape(-1), group)
    g = g.reshape(B, M * nsample, dpad)
    centp = jnp.concatenate(
        [cent, jnp.zeros((B, M, dpad - 3), cent.dtype)], axis=-1)
    wa = _conv_args(pp, pad_in=dpad)
    return cent, _mlp_max(g, centp, nsample, mb, wa)


def kernel(x, params):
    xyz = x[:, :, :3]
    feats = x[:, :, 3:]
    k1, k2 = jax.random.split(jax.random.key(42), 2)
    sel1 = jax.random.permutation(k1, 4096)[:512]
    sel2 = jax.random.permutation(k2, 512)[:128]

    xyz1, p1 = _sa_stage(xyz, feats, sel1, 32, params['sa1'],
                         dpad=16, mb=128, group=8)
    xyz2, p2 = _sa_stage(xyz1, p1, sel2, 64, params['sa2'],
                         dpad=144, mb=32, group=2)
    return _tail(p2, params)


# speedup vs baseline: 5.1539x; 1.0126x over previous
import functools

import jax
import jax.numpy as jnp
import numpy as np
from jax.experimental import pallas as pl
from jax.experimental.pallas import tpu as pltpu
from jax.experimental.pallas import tpu_sc as plsc

EPS = 1e-5
BF = jnp.bfloat16


def _knn_kernel(k, nbase, cent_ref, pts_ref, idx_ref, d_ref):
    M, N = d_ref.shape
    c = cent_ref[...]
    cx, cy, cz = c[:, 0:1], c[:, 1:2], c[:, 2:3]
    px, py, pz = pts_ref[0:1, :], pts_ref[1:2, :], pts_ref[2:3, :]
    cn2 = cx * cx + cy * cy + cz * cz
    pn2 = px * px + py * py + pz * pz

    def r(v):
        return v.astype(BF).astype(jnp.float32)
    cross = r(cx) * r(px) + r(cy) * r(py) + r(cz) * r(pz)
    d_ref[...] = (cn2 + pn2) - 2.0 * cross

    lane = jax.lax.broadcasted_iota(jnp.int32, (M, N), 1)
    olane = jax.lax.broadcasted_iota(jnp.int32, (M, k), 1)
    big_i = jnp.int32(2 ** 30)
    inf = jnp.float32(np.inf)
    base = pl.program_id(0) * nbase

    UNROLL = 8
    assert k % UNROLL == 0

    def body(it, _):
        d = d_ref[...]
        out = idx_ref[...]
        for u in range(UNROLL):
            rowmin = jnp.min(d, axis=1, keepdims=True)
            cand = jnp.where(d == rowmin, lane, big_i)
            amin = jnp.min(cand, axis=1, keepdims=True)
            out = jnp.where(olane == it * UNROLL + u, amin + base, out)
            d = jnp.where(cand == amin, inf, d)
        idx_ref[...] = out
        d_ref[...] = d
        return 0

    jax.lax.fori_loop(0, k // UNROLL, body, 0)


def _knn_idx(cent, pts, k):
    B, M, _ = cent.shape
    N = pts.shape[1]
    return pl.pallas_call(
        functools.partial(_knn_kernel, k, N),
        grid=(B,),
        in_specs=[
            pl.BlockSpec((None, M, 3), lambda b: (b, 0, 0)),
            pl.BlockSpec((None, 3, N), lambda b: (b, 0, 0)),
        ],
        out_specs=pl.BlockSpec((None, M, k), lambda b: (b, 0, 0)),
        out_shape=jax.ShapeDtypeStruct((B, M, k), jnp.int32),
        scratch_shapes=[pltpu.VMEM((M, N), jnp.float32)],
    )(cent, pts.transpose(0, 2, 1))


def _sc_gather(table, idx, group):
    V, D = table.shape
    R = idx.shape[0]
    NW = 32
    rpw = R // NW
    nch = rpw // 128
    ngrp = nch // group
    rows_g = group * 128
    idx2 = idx.reshape(NW, nch, 128)
    mesh = plsc.VectorSubcoreMesh(core_axis_name="c", subcore_axis_name="s")

    @functools.partial(
        pl.kernel,
        out_type=jax.ShapeDtypeStruct((R, D), jnp.float32),
        mesh=mesh,
        compiler_params=pltpu.CompilerParams(use_tc_tiling_on_sc=False),
        scratch_types=[
            pltpu.VMEM((nch, 128), jnp.int32),
            pltpu.VMEM((rows_g, D), jnp.float32),
            pltpu.VMEM((rows_g, D), jnp.float32),
            pltpu.SemaphoreType.DMA,
            pltpu.SemaphoreType.DMA,
        ],
    )
    def k(table_hbm, idx_hbm, out_hbm, idx_v, b0, b1, s0, s1):
        wid = jax.lax.axis_index("s") * 2 + jax.lax.axis_index("c")
        base = wid * rpw
        pltpu.sync_copy(idx_hbm.at[wid], idx_v)
        bufs = (b0, b1)
        sems = (s0, s1)
        pend = [None, None]

        def fire(gi, slot):
            cps = []
            for j in range(group):
                cps.append(pltpu.async_copy(
                    table_hbm.at[idx_v.at[gi * group + j]],
                    bufs[slot].at[pl.ds(j * 128, 128)],
                    sems[slot]))
            pend[slot] = cps

        def drain(gi, slot):
            for cp in pend[slot]:
                cp.wait()
            pltpu.sync_copy(bufs[slot],
                            out_hbm.at[pl.ds(base + gi * rows_g, rows_g)])

        fire(0, 0)
        for gi in range(1, ngrp):
            fire(gi, gi % 2)
            drain(gi - 1, (gi - 1) % 2)
        drain(ngrp - 1, (ngrp - 1) % 2)

    return k(table, idx2)


def _mlp_kernel(kn, g_ref, cent_ref, w1_ref, a1_ref, w2_ref, a2_ref,
                w3_ref, a3_ref, o_ref):
    MB = cent_ref.shape[0]
    D = g_ref.shape[1]
    t = g_ref[...] - jnp.broadcast_to(
        cent_ref[...][:, None, :], (MB, kn, D)).reshape(MB * kn, D)

    def layer(h, w_ref, a_ref, relu):
        h = jnp.dot(h.astype(BF), w_ref[...].astype(BF),
                    preferred_element_type=jnp.float32)
        h = h + a_ref[0:1, :]
        h = h * a_ref[1:2, :] + a_ref[2:3, :]
        return jnp.maximum(h, 0.0) if relu else h

    h = layer(t, w1_ref, a1_ref, True)
    h = layer(h, w2_ref, a2_ref, True)
    h = layer(h, w3_ref, a3_ref, False)
    C = h.shape[1]
    o_ref[...] = jnp.max(h.reshape(MB, kn, C), axis=1)


def _mlp_max(g, cent, kn, mb, wa):
    B, M, D = cent.shape
    C3 = wa[4].shape[1]
    nmb = M // mb
    specs = [
        pl.BlockSpec((None, mb * kn, D), lambda b, m: (b, m, 0)),
        pl.BlockSpec((None, mb, D), lambda b, m: (b, m, 0)),
    ]
    for w in wa:
        specs.append(pl.BlockSpec(w.shape, lambda b, m: (0,) * w.ndim))
    return pl.pallas_call(
        functools.partial(_mlp_kernel, kn),
        grid=(B, nmb),
        in_specs=specs,
        out_specs=pl.BlockSpec((None, mb, C3), lambda b, m: (b, m, 0)),
        out_shape=jax.ShapeDtypeStruct((B, M, C3), jnp.float32),
    )(g, cent, *wa)


def _tail_kernel(B, M, t_ref, w1_ref, a1_ref, w2_ref, a2_ref, w3_ref, a3_ref,
                 f1_ref, g1_ref, f2_ref, g2_ref, f3_ref, b3_ref, o_ref):
    def layer(h, w_ref, a_ref, relu):
        h = jnp.dot(h.astype(BF), w_ref[...].astype(BF),
                    preferred_element_type=jnp.float32)
        h = h + a_ref[0:1, :]
        h = h * a_ref[1:2, :] + a_ref[2:3, :]
        return jnp.maximum(h, 0.0) if relu else h

    h = layer(t_ref[...], w1_ref, a1_ref, True)
    h = layer(h, w2_ref, a2_ref, True)
    h = layer(h, w3_ref, a3_ref, False)
    C = h.shape[1]
    f = jnp.max(h.reshape(B, M, C), axis=1)

    def fc(h, w_ref, a_ref):
        h = jnp.dot(h.astype(BF), w_ref[...].astype(BF),
                    preferred_element_type=jnp.float32)
        h = h + a_ref[0:1, :]
        return jnp.maximum(h * a_ref[1:2, :] + a_ref[2:3, :], 0.0)

    f = fc(f, f1_ref, g1_ref)
    f = fc(f, f2_ref, g2_ref)
    o_ref[...] = jnp.dot(f.astype(BF), f3_ref[...].astype(BF),
                         preferred_element_type=jnp.float32) + b3_ref[...]


def _tail(p2, params):
    B, M, C = p2.shape
    t = jnp.concatenate([jnp.zeros((B, M, 3), p2.dtype), p2], axis=-1)
    t = t.reshape(B * M, C + 3)
    pp = params['sa3']
    wa = _conv_args(pp)
    s1 = params['bn1g'] / jnp.sqrt(1.0 + EPS)
    g1 = jnp.stack([params['fc1b'], s1, params['bn1b']])
    s2 = params['bn2g'] / jnp.sqrt(1.0 + EPS)
    g2 = jnp.stack([params['fc2b'], s2, params['bn2b']])
    return pl.pallas_call(
        functools.partial(_tail_kernel, B, M),
        out_shape=jax.ShapeDtypeStruct((B, 12), jnp.float32),
    )(t, wa[0], wa[1], wa[2], wa[3], wa[4], wa[5],
      params['fc1W'].T, g1, params['fc2W'].T, g2, params['fc3W'].T,
      params['fc3b'][None, :])


def _conv_args(pp, pad_in=None):
    out = []
    for i in (1, 2, 3):
        W = pp['W%d' % i].T
        if i == 1 and pad_in is not None and W.shape[0] < pad_in:
            W = jnp.concatenate(
                [W, jnp.zeros((pad_in - W.shape[0], W.shape[1]), W.dtype)], 0)
        s = pp['g%d' % i] / jnp.sqrt(1.0 + EPS)
        out.append(W)
        out.append(jnp.stack([pp['b%d' % i], s, pp['be%d' % i]]))
    return out


def _sa_stage(xyz, feats, sel, nsample, pp, dpad, mb, group):
    B, N, _ = xyz.shape
    cols = [xyz] if feats is None else [xyz, feats]
    cin = sum(c.shape[-1] for c in cols)
    if cin < dpad:
        cols.append(jnp.zeros((B, N, dpad - cin), xyz.dtype))
    pf = jnp.concatenate(cols, axis=-1)
    cent = jnp.take(xyz, sel, axis=1)
    M = cent.shape[1]
    idx = _knn_idx(cent, xyz, nsample)
    g = _sc_gather(pf.reshape(B * N, dpad), idx.reshape(-1), group)
    g = g.reshape(B, M * nsample, dpad)
    centp = jnp.concatenate(
        [cent, jnp.zeros((B, M, dpad - 3), cent.dtype)], axis=-1)
    wa = _conv_args(pp, pad_in=dpad)
    return cent, _mlp_max(g, centp, nsample, mb, wa)


def kernel(x, params):
    xyz = x[:, :, :3]
    feats = x[:, :, 3:]
    k1, k2 = jax.random.split(jax.random.key(42), 2)
    sel1 = jax.random.permutation(k1, 4096)[:512]
    sel2 = jax.random.permutation(k2, 512)[:128]

    xyz1, p1 = _sa_stage(xyz, feats, sel1, 32, params['sa1'],
                         dpad=16, mb=128, group=8)
    xyz2, p2 = _sa_stage(xyz1, p1, sel2, 64, params['sa2'],
                         dpad=144, mb=32, group=2)
    return _tail(p2, params)
